# Initial kernel scaffold; baseline (speedup 1.0000x reference)
import jax
import jax.numpy as jnp
from jax.experimental import pallas as pl


def kernel(x):
    raise NotImplementedError("write your pallas kernel here")


# speedup vs baseline: 13.9530x; 13.9530x over previous
import functools

import jax
import jax.numpy as jnp
from jax import lax
from jax.experimental import pallas as pl
from jax.experimental.pallas import tpu as pltpu
from jax.experimental.pallas import tpu_sc as plsc

B, C, H, W = 4, 3, 512, 512
NC, NS, L = 2, 16, 16
NWORK = NC * NS
RPT = H // NWORK
HB = RPT + 1
NCH = B * C
NBINS = 512
RES = H * W
NELEM = B * C * H * W
NSTEP = 19
WCH = W // L


def _tf_subg(r, g, b):
    return r - g, g, b - g


def _tf_jpeg2000(r, g, b):
    r2 = r - g
    b2 = b - g
    return r2, g + (r2 + b2) * 0.25, b2


def _tf_ycocg_r(r, g, b):
    r2 = r - b
    b1 = b + r2 * 0.5
    g2 = g - b1
    return r2, g2, b1 + g2 * 0.5


def _tf_ycbcr(final):
    def f(r, g, b):
        r2 = r - g
        g1 = g + r2 * 0.5
        b2 = b - g1
        return r2, g1 + final(r2, b2), b2
    return f


_TFS = [
    _tf_subg,
    _tf_jpeg2000,
    _tf_ycocg_r,
    _tf_ycbcr(lambda r, b: b * 0.5),
    _tf_ycbcr(lambda r, b: (2 * b - r) * 0.125),
    _tf_ycbcr(lambda r, b: (2 * b + r) * 0.125),
    _tf_ycbcr(lambda r, b: b / 3),
    _tf_ycbcr(lambda r, b: b * 0.375),
    _tf_ycbcr(lambda r, b: b * 0.4375),
]


def _fmod1(v):
    return lax.rem(v + 1.0, 2.0) - 1.0


def _sc_body(x_hbm, out_hbm, xbuf, hist):
    cid = lax.axis_index("core")
    sid = lax.axis_index("sub")
    wid = sid * NC + cid
    row0 = wid * RPT
    halo0 = jnp.maximum(row0 - 1, 0)

    for b in range(B):
        for c in range(C):
            pltpu.sync_copy(x_hbm.at[b, c, pl.ds(halo0, HB)],
                            xbuf.at[pl.ds((b * C + c) * HB, HB)])

    zero16 = jnp.zeros((L,), jnp.float32)
    ones16 = jnp.ones((L,), jnp.float32)
    iota16 = lax.iota(jnp.int32, L)

    @pl.when(wid == 0)
    def _zero_halo():
        def zrow(i, carry):
            bc = i // WCH
            j = i % WCH
            xbuf[bc * HB, pl.ds(j * L, L)] = zero16
            return carry
        lax.fori_loop(0, NCH * WCH, zrow, 0)

    for step in range(NSTEP):
        def zh(i, carry):
            hist[pl.ds(i * L, L)] = zero16
            return carry
        lax.fori_loop(0, (NCH * NBINS) // L, zh, 0)

        if step < 2 * len(_TFS):
            tf = _TFS[step // 2]

            def pa(i, carry):
                b = i // (HB * WCH)
                r_ = i % (HB * WCH)
                h = r_ // WCH
                j = r_ % WCH
                rrow = (b * C) * HB + h
                sl = pl.ds(j * L, L)
                r = xbuf[rrow, sl]
                g = xbuf[rrow + HB, sl]
                bb = xbuf[rrow + 2 * HB, sl]
                r2, g2, b2 = tf(r, g, bb)
                xbuf[rrow, sl] = r2
                xbuf[rrow + HB, sl] = g2
                xbuf[rrow + 2 * HB, sl] = b2
                return carry
            lax.fori_loop(0, B * HB * WCH, pa, 0)

            use_fmod = (step % 2 == 1)
            start = -1.0 if use_fmod else -2.0
            nbins = 256 if use_fmod else 512
        else:
            use_fmod = False
            start = -1.0
            nbins = 256
        scale = 128.0

        def pb(i, carry):
            bc = i // (RPT * WCH)
            r_ = i % (RPT * WCH)
            h = r_ // WCH + 1
            j = r_ % WCH
            rowc = bc * HB + h
            rowp = rowc - 1
            sl = pl.ds(j * L, L)
            cur = xbuf[rowc, sl]
            n = xbuf[rowp, sl]
            widx = j * L - 1 + iota16
            wcol = jnp.maximum(widx, 0)
            edge = widx >= 0
            w_ = plsc.load_gather(xbuf, [jnp.broadcast_to(rowc, (L,)), wcol])
            nw = plsc.load_gather(xbuf, [jnp.broadcast_to(rowp, (L,)), wcol])
            w_ = jnp.where(edge, w_, 0.0)
            nw = jnp.where(edge, nw, 0.0)
            if use_fmod:
                cur = _fmod1(cur)
                n = _fmod1(n)
                w_ = jnp.where(edge, _fmod1(w_), 0.0)
                nw = jnp.where(edge, _fmod1(nw), 0.0)
            pred = jnp.clip(n + w_ - nw, jnp.minimum(n, w_),
                            jnp.maximum(n, w_))
            d = lax.rem(cur - pred + 1.0, 2.0) - 1.0
            idx = jnp.clip(((d - start) * scale).astype(jnp.int32),
                           0, nbins - 1) + bc * NBINS
            m = (d >= start) & (d <= -start)
            plsc.addupdate_scatter(hist, [idx], ones16, mask=m)
            return carry
        lax.fori_loop(0, NCH * RPT * WCH, pb, 0)

        pltpu.sync_copy(hist, out_hbm.at[step, wid])


_sc_hist = functools.partial(
    pl.kernel,
    out_type=jax.ShapeDtypeStruct((NSTEP, NWORK, NCH * NBINS), jnp.float32),
    mesh=plsc.VectorSubcoreMesh(core_axis_name="core", subcore_axis_name="sub"),
    scratch_types=[
        pltpu.VMEM((NCH * HB, W), jnp.float32),
        pltpu.VMEM((NCH * NBINS,), jnp.float32),
    ],
    compiler_params=pltpu.CompilerParams(use_tc_tiling_on_sc=False,
                                         needs_layout_passes=False),
)(_sc_body)


def _ent_body(h_ref, o_ref):
    counts = jnp.sum(h_ref[...], axis=1)
    p = counts * (1.0 / RES)
    lp = jnp.log2(jnp.where(counts > 0, p, 1.0))
    ent = jnp.sum(p * lp, axis=1) * (-NELEM / (8.0 * NCH))
    o_ref[...] = lax.broadcast_in_dim(ent, (NSTEP, 128), (0,))


def kernel(x):
    part = _sc_hist(x)
    ent = pl.pallas_call(
        _ent_body,
        out_shape=jax.ShapeDtypeStruct((NSTEP, 128), jnp.float32),
    )(part)
    return ent[:, 0]


# speedup vs baseline: 42.5628x; 3.0504x over previous
import functools

import jax
import jax.numpy as jnp
from jax import lax
from jax.experimental import pallas as pl
from jax.experimental.pallas import tpu as pltpu
from jax.experimental.pallas import tpu_sc as plsc

B, C, H, W = 4, 3, 512, 512
NC, NS, L = 2, 16, 16
NWORK = NC * NS
RPT = H // NWORK
HB = RPT + 1
NCH = B * C
NBINS = 512
RES = H * W
NELEM = B * C * H * W
NSTEP = 19
WCH = W // L


def _tf_subg(r, g, b):
    return r - g, g, b - g


def _tf_jpeg2000(r, g, b):
    r2 = r - g
    b2 = b - g
    return r2, g + (r2 + b2) * 0.25, b2


def _tf_ycocg_r(r, g, b):
    r2 = r - b
    b1 = b + r2 * 0.5
    g2 = g - b1
    return r2, g2, b1 + g2 * 0.5


def _tf_ycbcr(final):
    def f(r, g, b):
        r2 = r - g
        g1 = g + r2 * 0.5
        b2 = b - g1
        return r2, g1 + final(r2, b2), b2
    return f


_TFS = [
    _tf_subg,
    _tf_jpeg2000,
    _tf_ycocg_r,
    _tf_ycbcr(lambda r, b: b * 0.5),
    _tf_ycbcr(lambda r, b: (2 * b - r) * 0.125),
    _tf_ycbcr(lambda r, b: (2 * b + r) * 0.125),
    _tf_ycbcr(lambda r, b: b / 3),
    _tf_ycbcr(lambda r, b: b * 0.375),
    _tf_ycbcr(lambda r, b: b * 0.4375),
]


def _fmod1(v):
    return lax.rem(v + 1.0, 2.0) - 1.0


def _sc_body(x_hbm, out_hbm, xbuf, hist):
    cid = lax.axis_index("core")
    sid = lax.axis_index("sub")
    wid = sid * NC + cid
    row0 = wid * RPT
    halo0 = jnp.maximum(row0 - 1, 0)

    for b in range(B):
        for c in range(C):
            pltpu.sync_copy(x_hbm.at[b, c, pl.ds(halo0, HB)],
                            xbuf.at[pl.ds(1 + (b * C + c) * HB, HB)])

    zero16 = jnp.zeros((L,), jnp.float32)
    ones16 = jnp.ones((L,), jnp.float32)
    iota16 = lax.iota(jnp.int32, L)

    @pl.when(wid == 0)
    def _zero_halo():
        def zrow(i, carry):
            bc = i // WCH
            j = i % WCH
            xbuf[1 + bc * HB, pl.ds(j * L, L)] = zero16
            return carry
        lax.fori_loop(0, NCH * WCH, zrow, 0)

    for step in range(NSTEP):
        @plsc.parallel_loop(0, (NCH * NBINS) // L, unroll=8)
        def zh(i):
            hist[pl.ds(i * L, L)] = zero16

        if step < 2 * len(_TFS):
            tf = _TFS[step // 2]

            @plsc.parallel_loop(0, B * HB * WCH, unroll=4)
            def pa(i):
                b = i // (HB * WCH)
                r_ = i % (HB * WCH)
                h = r_ // WCH
                j = r_ % WCH
                rrow = 1 + (b * C) * HB + h
                sl = pl.ds(j * L, L)
                r = xbuf[rrow, sl]
                g = xbuf[rrow + HB, sl]
                bb = xbuf[rrow + 2 * HB, sl]
                r2, g2, b2 = tf(r, g, bb)
                xbuf[rrow, sl] = r2
                xbuf[rrow + HB, sl] = g2
                xbuf[rrow + 2 * HB, sl] = b2

            use_fmod = (step % 2 == 1)
            start = -1.0 if use_fmod else -2.0
            nbins = 256 if use_fmod else 512
        else:
            use_fmod = False
            start = -1.0
            nbins = 256
        scale = 128.0

        @plsc.parallel_loop(0, NCH * RPT * WCH, unroll=4)
        def pb(i):
            bc = i // (RPT * WCH)
            r_ = i % (RPT * WCH)
            h = r_ // WCH + 1
            j = r_ % WCH
            rowc = 1 + bc * HB + h
            rowp = rowc - 1
            hbase = bc * NBINS
            s = j * L
            cur = xbuf[rowc, pl.ds(s, L)]
            n = xbuf[rowp, pl.ds(s, L)]
            w_ = xbuf[rowc, pl.ds(s - 1, L)]
            nw = xbuf[rowp, pl.ds(s - 1, L)]
            edge = (s - 1 + iota16) >= 0
            w_ = jnp.where(edge, w_, 0.0)
            nw = jnp.where(edge, nw, 0.0)
            if use_fmod:
                cur = _fmod1(cur)
                n = _fmod1(n)
                w_ = jnp.where(edge, _fmod1(w_), 0.0)
                nw = jnp.where(edge, _fmod1(nw), 0.0)
            pred = jnp.clip(n + w_ - nw, jnp.minimum(n, w_),
                            jnp.maximum(n, w_))
            d = lax.rem(cur - pred + 1.0, 2.0) - 1.0
            idx = jnp.clip(((d - start) * scale).astype(jnp.int32),
                           0, nbins - 1) + hbase
            m = (d >= start) & (d <= -start)
            plsc.addupdate_scatter(hist, [idx], ones16, mask=m)

        pltpu.sync_copy(hist, out_hbm.at[step, wid])


_sc_hist = functools.partial(
    pl.kernel,
    out_type=jax.ShapeDtypeStruct((NSTEP, NWORK, NCH * NBINS), jnp.float32),
    mesh=plsc.VectorSubcoreMesh(core_axis_name="core", subcore_axis_name="sub"),
    scratch_types=[
        pltpu.VMEM((1 + NCH * HB, W), jnp.float32),
        pltpu.VMEM((NCH * NBINS,), jnp.float32),
    ],
    compiler_params=pltpu.CompilerParams(use_tc_tiling_on_sc=False,
                                         needs_layout_passes=False),
)(_sc_body)


def _ent_body(h_ref, o_ref):
    counts = jnp.sum(h_ref[...], axis=1)
    p = counts * (1.0 / RES)
    lp = jnp.log2(jnp.where(counts > 0, p, 1.0))
    ent = jnp.sum(p * lp, axis=1) * (-NELEM / (8.0 * NCH))
    o_ref[...] = lax.broadcast_in_dim(ent, (NSTEP, 128), (0,))


def kernel(x):
    part = _sc_hist(x)
    ent = pl.pallas_call(
        _ent_body,
        out_shape=jax.ShapeDtypeStruct((NSTEP, 128), jnp.float32),
    )(part)
    return ent[:, 0]


# speedup vs baseline: 70.2444x; 1.6504x over previous
import functools

import jax
import jax.numpy as jnp
from jax import lax
from jax.experimental import pallas as pl
from jax.experimental.pallas import tpu as pltpu
from jax.experimental.pallas import tpu_sc as plsc

B, C, H, W = 4, 3, 512, 512
NC, NS, L = 2, 16, 16
NWORK = NC * NS
RPT = H // NWORK
HB = RPT + 1
NCH = B * C
NBINS = 512
RES = H * W
NELEM = B * C * H * W
NSTEP = 19
WCH = W // L


def _tf_subg(r, g, b):
    return r - g, g, b - g


def _tf_jpeg2000(r, g, b):
    r2 = r - g
    b2 = b - g
    return r2, g + (r2 + b2) * 0.25, b2


def _tf_ycocg_r(r, g, b):
    r2 = r - b
    b1 = b + r2 * 0.5
    g2 = g - b1
    return r2, g2, b1 + g2 * 0.5


def _tf_ycbcr(final):
    def f(r, g, b):
        r2 = r - g
        g1 = g + r2 * 0.5
        b2 = b - g1
        return r2, g1 + final(r2, b2), b2
    return f


_TFS = [
    _tf_subg,
    _tf_jpeg2000,
    _tf_ycocg_r,
    _tf_ycbcr(lambda r, b: b * 0.5),
    _tf_ycbcr(lambda r, b: (2 * b - r) * 0.125),
    _tf_ycbcr(lambda r, b: (2 * b + r) * 0.125),
    _tf_ycbcr(lambda r, b: b / 3),
    _tf_ycbcr(lambda r, b: b * 0.375),
    _tf_ycbcr(lambda r, b: b * 0.4375),
]


def _trem2(a):
    t = (a * 0.5).astype(jnp.int32).astype(jnp.float32)
    return a - (t + t)


def _fmod1(v):
    return _trem2(v + 1.0) - 1.0


def _resid(cur, n, w_, nw):
    pred = jnp.clip(n + w_ - nw, jnp.minimum(n, w_), jnp.maximum(n, w_))
    return cur - pred + 1.0


def _bin256(a):
    r = _trem2(a)
    return jnp.maximum((r * 128.0).astype(jnp.int32), 0), r >= 0.0


def _bin512(a):
    r = _trem2(a)
    return jnp.maximum(((r + 1.0) * 128.0).astype(jnp.int32), 0), r >= -1.0


def _sc_body(x_hbm, out_hbm, xbuf, hist, ubuf):
    cid = lax.axis_index("core")
    sid = lax.axis_index("sub")
    wid = sid * NC + cid
    row0 = wid * RPT
    halo0 = jnp.maximum(row0 - 1, 0)

    for b in range(B):
        for c in range(C):
            pltpu.sync_copy(x_hbm.at[b, c, pl.ds(halo0, HB)],
                            xbuf.at[pl.ds(1 + (b * C + c) * HB, HB)])

    zero16 = jnp.zeros((L,), jnp.float32)
    ones16 = jnp.ones((L,), jnp.float32)
    iota16 = lax.iota(jnp.int32, L)

    @pl.when(wid == 0)
    def _zero_halo():
        def zrow(i, carry):
            bc = i // WCH
            j = i % WCH
            xbuf[1 + bc * HB, pl.ds(j * L, L)] = zero16
            return carry
        lax.fori_loop(0, NCH * WCH, zrow, 0)

    for step in range(NSTEP):
        @plsc.parallel_loop(0, (NCH * NBINS) // L, unroll=8)
        def zh(i):
            hist[pl.ds(i * L, L)] = zero16

        if step < 2 * len(_TFS):
            tf = _TFS[step // 2]

            @plsc.parallel_loop(0, B * HB * WCH, unroll=4)
            def pa(i):
                b = i // (HB * WCH)
                r_ = i % (HB * WCH)
                h = r_ // WCH
                j = r_ % WCH
                rrow = 1 + (b * C) * HB + h
                sl = pl.ds(j * L, L)
                r = xbuf[rrow, sl]
                g = xbuf[rrow + HB, sl]
                bb = xbuf[rrow + 2 * HB, sl]
                r2, g2, b2 = tf(r, g, bb)
                xbuf[rrow, sl] = r2
                xbuf[rrow + HB, sl] = g2
                xbuf[rrow + 2 * HB, sl] = b2

        use_fmod = step < 2 * len(_TFS) and step % 2 == 1
        binf = _bin256 if (use_fmod or step == NSTEP - 1) else _bin512
        lane_ok = iota16 >= 1

        if use_fmod:
            def fb(bc, carry):
                xrow0 = 1 + bc * HB
                hbase = bc * NBINS

                @plsc.parallel_loop(0, HB * WCH, unroll=4)
                def up(i):
                    h = i // WCH
                    j = i % WCH
                    sl = pl.ds(j * L, L)
                    ubuf[1 + h, sl] = _fmod1(xbuf[xrow0 + h, sl])

                @plsc.parallel_loop(0, RPT * (WCH - 1), unroll=4)
                def hi(i):
                    h = i // (WCH - 1)
                    j = i % (WCH - 1) + 1
                    rowc = 2 + h
                    s = j * L
                    cur = ubuf[rowc, pl.ds(s, L)]
                    n = ubuf[rowc - 1, pl.ds(s, L)]
                    w_ = ubuf[rowc, pl.ds(s - 1, L)]
                    nw = ubuf[rowc - 1, pl.ds(s - 1, L)]
                    idx, m = binf(_resid(cur, n, w_, nw))
                    plsc.addupdate_scatter(hist, [idx + hbase], ones16,
                                           mask=m)

                @plsc.parallel_loop(0, RPT, unroll=2)
                def he(h):
                    rowc = 2 + h
                    cur = ubuf[rowc, pl.ds(0, L)]
                    n = ubuf[rowc - 1, pl.ds(0, L)]
                    w_ = jnp.where(lane_ok, ubuf[rowc, pl.ds(-1, L)], 0.0)
                    nw = jnp.where(lane_ok, ubuf[rowc - 1, pl.ds(-1, L)],
                                   0.0)
                    idx, m = binf(_resid(cur, n, w_, nw))
                    plsc.addupdate_scatter(hist, [idx + hbase], ones16,
                                           mask=m)
                return carry
            lax.fori_loop(0, NCH, fb, 0)
        else:
            @plsc.parallel_loop(0, NCH * RPT * (WCH - 1), unroll=4)
            def pbi(i):
                bc = i // (RPT * (WCH - 1))
                r_ = i % (RPT * (WCH - 1))
                h = r_ // (WCH - 1)
                j = r_ % (WCH - 1) + 1
                rowc = 2 + bc * HB + h
                s = j * L
                cur = xbuf[rowc, pl.ds(s, L)]
                n = xbuf[rowc - 1, pl.ds(s, L)]
                w_ = xbuf[rowc, pl.ds(s - 1, L)]
                nw = xbuf[rowc - 1, pl.ds(s - 1, L)]
                idx, m = binf(_resid(cur, n, w_, nw))
                plsc.addupdate_scatter(hist, [idx + bc * NBINS], ones16,
                                       mask=m)

            @plsc.parallel_loop(0, NCH * RPT, unroll=2)
            def pbe(i):
                bc = i // RPT
                h = i % RPT
                rowc = 2 + bc * HB + h
                cur = xbuf[rowc, pl.ds(0, L)]
                n = xbuf[rowc - 1, pl.ds(0, L)]
                w_ = jnp.where(lane_ok, xbuf[rowc, pl.ds(-1, L)], 0.0)
                nw = jnp.where(lane_ok, xbuf[rowc - 1, pl.ds(-1, L)], 0.0)
                idx, m = binf(_resid(cur, n, w_, nw))
                plsc.addupdate_scatter(hist, [idx + bc * NBINS], ones16,
                                       mask=m)

        pltpu.sync_copy(hist, out_hbm.at[step, wid])


_sc_hist = functools.partial(
    pl.kernel,
    out_type=jax.ShapeDtypeStruct((NSTEP, NWORK, NCH * NBINS), jnp.float32),
    mesh=plsc.VectorSubcoreMesh(core_axis_name="core", subcore_axis_name="sub"),
    scratch_types=[
        pltpu.VMEM((1 + NCH * HB, W), jnp.float32),
        pltpu.VMEM((NCH * NBINS,), jnp.float32),
        pltpu.VMEM((HB + 1, W), jnp.float32),
    ],
    compiler_params=pltpu.CompilerParams(use_tc_tiling_on_sc=False,
                                         needs_layout_passes=False),
)(_sc_body)


def _ent_body(h_ref, o_ref):
    counts = jnp.sum(h_ref[...], axis=1)
    p = counts * (1.0 / RES)
    lp = jnp.log2(jnp.where(counts > 0, p, 1.0))
    ent = jnp.sum(p * lp, axis=1) * (-NELEM / (8.0 * NCH))
    o_ref[...] = lax.broadcast_in_dim(ent, (NSTEP, 128), (0,))


def kernel(x):
    part = _sc_hist(x)
    ent = pl.pallas_call(
        _ent_body,
        out_shape=jax.ShapeDtypeStruct((NSTEP, 128), jnp.float32),
    )(part)
    return ent[:, 0]
